# baseline (speedup 1.0000x reference)
import functools

import jax
import jax.numpy as jnp
from jax import lax
from jax.experimental import pallas as pl
from jax.experimental.pallas import tpu as pltpu
from jax.experimental.pallas import tpu_sc as plsc

CUT_NUM = 20
HIDDEN = 1024
EMBED_SCALE = 0.1
B = 1024
S = 256

NC = 2
NS = 16
NW = NC * NS
B_PER_W = B // NW

BB = 8


def _make_sc_gather():
    mesh = plsc.VectorSubcoreMesh(core_axis_name="c", subcore_axis_name="s")

    @functools.partial(
        pl.kernel,
        mesh=mesh,
        out_type=[
            jax.ShapeDtypeStruct((B, HIDDEN), jnp.float32),
            jax.ShapeDtypeStruct((B, HIDDEN), jnp.float32),
        ],
        scratch_types=[
            pltpu.VMEM((B_PER_W,), jnp.int32),
            pltpu.VMEM((B_PER_W,), jnp.int32),
            pltpu.VMEM((B_PER_W, HIDDEN), jnp.float32),
            pltpu.VMEM((B_PER_W, HIDDEN), jnp.float32),
            pltpu.SemaphoreType.DMA,
        ],
    )
    def sc_gather(p0_hbm, p1_hbm, h_table_hbm, w_table_hbm,
                  h_out, w_out, idx0, idx1, hrows, wrows, sem):
        wid = lax.axis_index("s") * NC + lax.axis_index("c")
        base = wid * B_PER_W
        pltpu.sync_copy(p0_hbm.at[pl.ds(base, B_PER_W)], idx0)
        pltpu.sync_copy(p1_hbm.at[pl.ds(base, B_PER_W)], idx1)
        pltpu.async_copy(h_table_hbm.at[idx0], hrows, sem).wait()
        pltpu.sync_copy(hrows, h_out.at[pl.ds(base, B_PER_W)])
        pltpu.async_copy(w_table_hbm.at[idx1], wrows, sem).wait()
        pltpu.sync_copy(wrows, w_out.at[pl.ds(base, B_PER_W)])

    return sc_gather


_sc_gather_cache = []


def _get_sc_gather():
    if not _sc_gather_cache:
        _sc_gather_cache.append(_make_sc_gather())
    return _sc_gather_cache[0]


def _add_body(hid_ref, h_ref, w_ref, out_ref):
    i = pl.program_id(0)
    sl = pl.ds(i * BB, BB)
    emb = (h_ref[sl, :] + w_ref[sl, :]) * EMBED_SCALE
    out_ref[...] = hid_ref[...] + emb[:, None, :]


def _make_half_add(offset, nb):
    def body(hid_ref, h_ref, w_ref, out_ref):
        i = pl.program_id(0)
        sl = pl.ds((offset + i) * BB, BB)
        emb = (h_ref[sl, :] + w_ref[sl, :]) * EMBED_SCALE
        out_ref[...] = hid_ref[...] + emb[:, None, :]

    return pl.pallas_call(
        body,
        grid=(nb,),
        in_specs=[
            pl.BlockSpec((BB, S, HIDDEN), lambda i: (offset + i, 0, 0)),
            pl.BlockSpec((B, HIDDEN), lambda i: (0, 0)),
            pl.BlockSpec((B, HIDDEN), lambda i: (0, 0)),
        ],
        out_specs=pl.BlockSpec((BB, S, HIDDEN), lambda i: (i, 0, 0)),
        out_shape=jax.ShapeDtypeStruct((nb * BB, S, HIDDEN), jnp.float32),
    )


_add_lo = _make_half_add(0, B // BB // 2)
_add_hi = _make_half_add(B // BB // 2, B // BB // 2)


def kernel(hidden_states, patch_positions, h_table, w_table):
    pp = patch_positions.astype(jnp.int32)
    p0 = pp[:, 0]
    p1 = pp[:, 1]
    h_emb, w_emb = _get_sc_gather()(p0, p1, h_table, w_table)
    lo = _add_lo(hidden_states, h_emb, w_emb)
    hi = _add_hi(hidden_states, h_emb, w_emb)
    return jnp.concatenate([lo, hi], axis=0)


# speedup vs baseline: 1.9786x; 1.9786x over previous
import functools

import jax
import jax.numpy as jnp
from jax import lax
from jax.experimental import pallas as pl
from jax.experimental.pallas import tpu as pltpu
from jax.experimental.pallas import tpu_sc as plsc

CUT_NUM = 20
HIDDEN = 1024
EMBED_SCALE = 0.1
B = 1024
S = 256

NC = 2
NS = 16
NW = NC * NS
B_PER_W = B // NW

BB = 8


def _make_sc_gather():
    mesh = plsc.VectorSubcoreMesh(core_axis_name="c", subcore_axis_name="s")

    @functools.partial(
        pl.kernel,
        mesh=mesh,
        out_type=jax.ShapeDtypeStruct((2 * B, HIDDEN), jnp.float32),
        scratch_types=[
            pltpu.VMEM((2 * B_PER_W,), jnp.int32),
            pltpu.VMEM((2 * B_PER_W, HIDDEN), jnp.float32),
            pltpu.SemaphoreType.DMA,
        ],
    )
    def sc_gather(idx_hbm, cat_table_hbm, emb_out, idx, rows, sem):
        wid = lax.axis_index("s") * NC + lax.axis_index("c")
        nrows = 2 * B_PER_W
        base = wid * nrows
        pltpu.sync_copy(idx_hbm.at[pl.ds(base, nrows)], idx)
        pltpu.async_copy(cat_table_hbm.at[idx], rows, sem).wait()
        pltpu.sync_copy(rows, emb_out.at[pl.ds(base, nrows)])

    return sc_gather


_sc_gather_cache = []


def _get_sc_gather():
    if not _sc_gather_cache:
        _sc_gather_cache.append(_make_sc_gather())
    return _sc_gather_cache[0]


def _add_body(hid_ref, emb_ref, out_ref):
    i = pl.program_id(0)
    h = emb_ref[pl.ds(i * BB, BB), :]
    w = emb_ref[pl.ds(B + i * BB, BB), :]
    emb = (h + w) * EMBED_SCALE
    out_ref[...] = hid_ref[...] + emb[:, None, :]


_broadcast_add = pl.pallas_call(
    _add_body,
    grid=(B // BB,),
    in_specs=[
        pl.BlockSpec((BB, S, HIDDEN), lambda i: (i, 0, 0)),
        pl.BlockSpec((2 * B, HIDDEN), lambda i: (0, 0)),
    ],
    out_specs=pl.BlockSpec((BB, S, HIDDEN), lambda i: (i, 0, 0)),
    out_shape=jax.ShapeDtypeStruct((B, S, HIDDEN), jnp.float32),
)


def kernel(hidden_states, patch_positions, h_table, w_table):
    pp = patch_positions.astype(jnp.int32)
    idx_all = jnp.concatenate([pp[:, 0], pp[:, 1] + CUT_NUM])
    cat_table = jnp.concatenate([h_table, w_table], axis=0)
    emb = _get_sc_gather()(idx_all, cat_table)
    return _broadcast_add(hidden_states, emb)
